# baseline (speedup 1.0000x reference)
import functools

import jax
import jax.numpy as jnp
from jax import lax
from jax.experimental import pallas as pl
from jax.experimental.pallas import tpu as pltpu
from jax.experimental.pallas import tpu_sc as plsc

_N = 2 * 8192 * 2048
_NW = 32
_PER_W = _N // _NW
_CH = 32768
_NCH = _PER_W // _CH


def _sc_body(x_hbm, v_hbm, out_hbm, buf0, buf1, vbuf, rs0, rs1, ws0, ws1):
    wid = lax.axis_index("s") * 2 + lax.axis_index("c")
    base = wid * _PER_W

    pltpu.sync_copy(v_hbm, vbuf)
    inv = 1.0 / vbuf[...]

    bufs = (buf0, buf1)
    rsems = (rs0, rs1)
    wsems = (ws0, ws1)

    reads = [None] * _NCH
    writes = [None] * _NCH
    reads[0] = pltpu.async_copy(x_hbm.at[pl.ds(base, _CH)], bufs[0], rsems[0])
    for g in range(_NCH):
        cur = bufs[g % 2]
        if g + 1 < _NCH:
            if g - 1 >= 0:
                writes[g - 1].wait()
            reads[g + 1] = pltpu.async_copy(
                x_hbm.at[pl.ds(base + (g + 1) * _CH, _CH)],
                bufs[(g + 1) % 2], rsems[(g + 1) % 2])
        reads[g].wait()

        def _mul(i, _, cur=cur, inv=inv):
            off = i * 16
            cur[pl.ds(off, 16)] = cur[pl.ds(off, 16)] * inv
            return 0

        lax.fori_loop(0, _CH // 16, _mul, 0)
        writes[g] = pltpu.async_copy(
            cur, out_hbm.at[pl.ds(base + g * _CH, _CH)], wsems[g % 2])
    writes[_NCH - 2].wait()
    writes[_NCH - 1].wait()


@functools.partial(jax.jit, static_argnames=())
def _scale_sc(xf, vf):
    mesh = plsc.VectorSubcoreMesh(core_axis_name="c", subcore_axis_name="s")
    f = pl.kernel(
        _sc_body,
        mesh=mesh,
        out_type=jax.ShapeDtypeStruct((_N,), jnp.float32),
        scratch_types=[
            pltpu.VMEM((_CH,), jnp.float32),
            pltpu.VMEM((_CH,), jnp.float32),
            pltpu.VMEM((16,), jnp.float32),
            pltpu.SemaphoreType.DMA,
            pltpu.SemaphoreType.DMA,
            pltpu.SemaphoreType.DMA,
            pltpu.SemaphoreType.DMA,
        ],
    )
    return f(xf, vf)


def kernel(x, value):
    b, s, d = x.shape
    xf = x.reshape(_N)
    vf = jnp.broadcast_to(value, (16,))
    out = _scale_sc(xf, vf)
    return out.reshape(b, s, d)


def _scale_body_tc(v_ref, x_ref, o_ref):
    o_ref[...] = x_ref[...] * (1.0 / v_ref[0, 0])


def _kernel_tc(x, value):
    b, s, d = x.shape
    rows = b * s
    xf = x.reshape(rows, d)
    vf = value.reshape(1, 1)

    blk = 1024
    grid = rows // blk

    out = pl.pallas_call(
        _scale_body_tc,
        grid=(grid,),
        in_specs=[
            pl.BlockSpec((1, 1), lambda i: (0, 0)),
            pl.BlockSpec((blk, d), lambda i: (i, 0)),
        ],
        out_specs=pl.BlockSpec((blk, d), lambda i: (i, 0)),
        out_shape=jax.ShapeDtypeStruct((rows, d), x.dtype),
    )(vf, xf)
    return out.reshape(b, s, d)
